# baseline (speedup 1.0000x reference)
import jax
import jax.numpy as jnp
from jax import lax
from jax.experimental import pallas as pl
from jax.experimental.pallas import tpu as pltpu
from jax.experimental.pallas import tpu_sc as plsc

_DMODEL = 64
_BLANES = 32
_BSUB = _DMODEL // _BLANES
_NC = 2
_NS = 16
_NW = _NC * _NS
_GROUP = 128
_KG = 4
_CHUNK = _KG * _GROUP
_SCALE = 8.0


def _sc_embedding_body(idx_hbm, table_hbm, out_hbm,
                       idx_v, rows0, rows1, sg0, sg1, so0, so1):
    n_grp = idx_hbm.shape[0]
    per_w_grp = n_grp // _NW
    n_chunks = per_w_grp // _KG
    wid = lax.axis_index("s") * _NC + lax.axis_index("c")
    g0 = wid * per_w_grp

    rows = (rows0, rows1)
    sg = (sg0, sg1)
    so = (so0, so1)

    def issue_gathers(chunk, buf, sem):
        for j in range(_KG):
            pltpu.async_copy(
                table_hbm.at[idx_v.at[chunk * _KG + j]],
                buf.at[pl.ds(j * _GROUP, _GROUP)],
                sem,
            )

    def drain_gathers(buf, sem):
        for j in range(_KG):
            pltpu.make_async_copy(
                table_hbm.at[idx_v.at[j]],
                buf.at[pl.ds(j * _GROUP, _GROUP)],
                sem,
            ).wait()

    def scale(buf):
        @plsc.parallel_loop(0, _CHUNK, unroll=8)
        def _scale(r):
            for s in range(_BSUB):
                sl = pl.ds(s * _BLANES, _BLANES)
                buf[r, sl] = buf[r, sl] * jnp.bfloat16(_SCALE)

    def writeback(chunk, buf, sem):
        return pltpu.async_copy(
            buf, out_hbm.at[pl.ds((g0 + chunk * _KG) * _GROUP, _CHUNK)], sem)

    pltpu.sync_copy(idx_hbm.at[pl.ds(g0, per_w_grp)], idx_v)

    issue_gathers(0, rows0, sg0)
    issue_gathers(1, rows1, sg1)

    @pl.loop(0, n_chunks - 2, step=2)
    def _steady(c):
        for b in range(2):
            cur = c + b
            drain_gathers(rows[b], sg[b])
            scale(rows[b])
            wb = writeback(cur, rows[b], so[b])
            wb.wait()
            issue_gathers(cur + 2, rows[b], sg[b])

    for b in range(2):
        cur = n_chunks - 2 + b
        drain_gathers(rows[b], sg[b])
        scale(rows[b])
        writeback(cur, rows[b], so[b]).wait()


@jax.jit
def kernel(token_ids, embedding_table):
    b0, b1 = token_ids.shape
    batch = b0 * b1
    n_grp = batch // _GROUP
    idx2d = token_ids.reshape(n_grp, _GROUP).astype(jnp.int32)
    table16 = embedding_table.astype(jnp.bfloat16)

    mesh = plsc.VectorSubcoreMesh(
        core_axis_name="c", subcore_axis_name="s",
        num_cores=_NC, num_subcores=_NS,
    )
    out = pl.kernel(
        _sc_embedding_body,
        out_type=jax.ShapeDtypeStruct((batch, _DMODEL), jnp.bfloat16),
        mesh=mesh,
        compiler_params=pltpu.CompilerParams(use_tc_tiling_on_sc=False),
        scratch_types=[
            pltpu.VMEM((n_grp // _NW, _GROUP), jnp.int32),
            pltpu.VMEM((_CHUNK, _DMODEL), jnp.bfloat16),
            pltpu.VMEM((_CHUNK, _DMODEL), jnp.bfloat16),
            pltpu.SemaphoreType.DMA,
            pltpu.SemaphoreType.DMA,
            pltpu.SemaphoreType.DMA,
            pltpu.SemaphoreType.DMA,
        ],
    )(idx2d, table16)
    return out.astype(jnp.float32).reshape(b0, b1, _DMODEL)


# speedup vs baseline: 1.4542x; 1.4542x over previous
import jax
import jax.numpy as jnp
from jax import lax
from jax.experimental import pallas as pl
from jax.experimental.pallas import tpu as pltpu
from jax.experimental.pallas import tpu_sc as plsc

_DMODEL = 64
_LANES = 16
_NSUB = _DMODEL // _LANES
_NC = 2
_NS = 16
_NW = _NC * _NS
_GROUP = 128
_KG = 2
_CHUNK = _KG * _GROUP
_NBUF = 4
_SCALE = 8.0


def _sc_embedding_body(idx_hbm, table_hbm, out_hbm, idx_v,
                       rows0, rows1, rows2, rows3,
                       sg0, sg1, sg2, sg3, so0, so1, so2, so3):
    n_grp = idx_hbm.shape[0]
    per_w_grp = n_grp // _NW
    n_chunks = per_w_grp // _KG
    wid = lax.axis_index("s") * _NC + lax.axis_index("c")
    g0 = wid * per_w_grp

    rows = (rows0, rows1, rows2, rows3)
    sg = (sg0, sg1, sg2, sg3)
    so = (so0, so1, so2, so3)

    def issue_gathers(chunk, b):
        for j in range(_KG):
            pltpu.async_copy(
                table_hbm.at[idx_v.at[chunk * _KG + j]],
                rows[b].at[pl.ds(j * _GROUP, _GROUP)],
                sg[b],
            )

    def drain_gathers(b):
        for j in range(_KG):
            pltpu.make_async_copy(
                table_hbm.at[idx_v.at[j]],
                rows[b].at[pl.ds(j * _GROUP, _GROUP)],
                sg[b],
            ).wait()

    def scale(b):
        buf = rows[b]

        @plsc.parallel_loop(0, _CHUNK, unroll=8)
        def _scale(r):
            for s in range(_NSUB):
                sl = pl.ds(s * _LANES, _LANES)
                buf[r, sl] = buf[r, sl] * jnp.float32(_SCALE)

    def issue_writeback(chunk, b):
        pltpu.async_copy(
            rows[b], out_hbm.at[pl.ds((g0 + chunk * _KG) * _GROUP, _CHUNK)],
            so[b])

    def wait_writeback(b):
        pltpu.make_async_copy(
            rows[b], out_hbm.at[pl.ds(0, _CHUNK)], so[b]).wait()

    pltpu.sync_copy(idx_hbm.at[pl.ds(g0, per_w_grp)], idx_v)

    for c in range(_NBUF - 1):
        issue_gathers(c, c)

    drain_gathers(0)
    scale(0)
    issue_writeback(0, 0)
    issue_gathers(_NBUF - 1, _NBUF - 1)

    @pl.loop(1, n_chunks - (_NBUF - 1), step=_NBUF)
    def _steady(c):
        for u in range(_NBUF):
            cur = c + u
            b = (1 + u) % _NBUF
            drain_gathers(b)
            scale(b)
            issue_writeback(cur, b)
            nb = (b + _NBUF - 1) % _NBUF
            wait_writeback(nb)
            issue_gathers(cur + _NBUF - 1, nb)

    for u in range(_NBUF - 1):
        cur = n_chunks - (_NBUF - 1) + u
        b = cur % _NBUF
        drain_gathers(b)
        scale(b)
        issue_writeback(cur, b)
    for b in range(_NBUF):
        wait_writeback(b)


@jax.jit
def kernel(token_ids, embedding_table):
    b0, b1 = token_ids.shape
    batch = b0 * b1
    n_grp = batch // _GROUP
    idx2d = token_ids.reshape(n_grp, _GROUP).astype(jnp.int32)

    mesh = plsc.VectorSubcoreMesh(
        core_axis_name="c", subcore_axis_name="s",
        num_cores=_NC, num_subcores=_NS,
    )
    out = pl.kernel(
        _sc_embedding_body,
        out_type=jax.ShapeDtypeStruct((batch, _DMODEL), jnp.float32),
        mesh=mesh,
        compiler_params=pltpu.CompilerParams(use_tc_tiling_on_sc=False),
        scratch_types=[
            pltpu.VMEM((n_grp // _NW, _GROUP), jnp.int32),
            pltpu.VMEM((_CHUNK, _DMODEL), jnp.float32),
            pltpu.VMEM((_CHUNK, _DMODEL), jnp.float32),
            pltpu.VMEM((_CHUNK, _DMODEL), jnp.float32),
            pltpu.VMEM((_CHUNK, _DMODEL), jnp.float32),
            pltpu.SemaphoreType.DMA,
            pltpu.SemaphoreType.DMA,
            pltpu.SemaphoreType.DMA,
            pltpu.SemaphoreType.DMA,
            pltpu.SemaphoreType.DMA,
            pltpu.SemaphoreType.DMA,
            pltpu.SemaphoreType.DMA,
            pltpu.SemaphoreType.DMA,
        ],
    )(idx2d, embedding_table)
    return out.reshape(b0, b1, _DMODEL)
